# baseline (device time: 50416 ns/iter reference)
import jax
import jax.numpy as jnp
from jax import lax
from jax.experimental import pallas as pl
from jax.experimental.pallas import tpu as pltpu

N_DEV = 8


def kernel(x, w_mat):
    m_total, k_per = x.shape
    k_total, n = w_mat.shape
    m_per = m_total // N_DEV

    def body(
        x_hbm, w_hbm, out_ref,
        x_vm, x_bf, w_vm, recv_buf,
        send_sems, recv_sems, ready_sems, sem_x, sem_w,
    ):
        my = lax.axis_index("i")
        my4 = lax.rem(my, 4)
        base = my - my4
        obase = 4 - base

        barrier_sem = pltpu.get_barrier_semaphore()
        pl.semaphore_signal(
            barrier_sem, inc=1,
            device_id=(my,), device_id_type=pl.DeviceIdType.MESH,
        )
        pl.semaphore_wait(barrier_sem, 1)

        cp_x = pltpu.make_async_copy(x_hbm, x_vm, sem_x)
        cp_x.start()
        cp_w = pltpu.make_async_copy(w_hbm, w_vm, sem_w)
        cp_w.start()

        for t in range(1, 4):
            q = base + lax.rem(my4 + t, 4)
            pl.semaphore_signal(
                ready_sems.at[3 - t], inc=1,
                device_id=(q,), device_id_type=pl.DeviceIdType.MESH,
            )
        for t in range(4):
            q = obase + lax.rem(my4 + t, 4)
            pl.semaphore_signal(
                ready_sems.at[3 + (4 - t) % 4], inc=1,
                device_id=(q,), device_id_type=pl.DeviceIdType.MESH,
            )

        cp_x.wait()
        x_bf[:, :] = x_vm[:, :].astype(jnp.bfloat16)

        rdmas = {}

        def send_to(q, dst_slot, gate_slot):
            pl.semaphore_wait(ready_sems.at[gate_slot], 1)
            rdma = pltpu.make_async_remote_copy(
                src_ref=x_bf.at[pl.ds(q * m_per, m_per), :],
                dst_ref=recv_buf.at[dst_slot],
                send_sem=send_sems.at[dst_slot],
                recv_sem=recv_sems.at[dst_slot],
                device_id=(q,),
                device_id_type=pl.DeviceIdType.MESH,
            )
            rdma.start()
            rdmas[dst_slot] = rdma

        for t in range(1, 4):
            q = base + lax.rem(my4 + t, 4)
            send_to(q, 3 - t, t - 1)
        for t in range(4):
            q = obase + lax.rem(my4 + t, 4)
            send_to(q, 3 + (4 - t) % 4, 3 + t)

        cp_w.wait()
        out_ref[:, :] = jnp.dot(
            x_bf[pl.ds(my * m_per, m_per), :],
            w_vm[pl.ds(my * k_per, k_per), :].astype(jnp.bfloat16),
            preferred_element_type=jnp.float32,
        )

        for s in range(N_DEV - 1):
            if s < 3:
                src = base + lax.rem(my4 + s + 1, 4)
            else:
                src = obase + lax.rem(my4 + (s - 3), 4)
            rdmas[s].wait_recv()
            out_ref[:, :] += jnp.dot(
                recv_buf[s],
                w_vm[pl.ds(src * k_per, k_per), :].astype(jnp.bfloat16),
                preferred_element_type=jnp.float32,
            )

        for s in range(N_DEV - 1):
            rdmas[s].wait_send()

        y = out_ref[:, :]
        out_ref[:, :] = y / (1.0 + jnp.exp(-y))

    return pl.pallas_call(
        body,
        out_shape=jax.ShapeDtypeStruct((m_per, n), jnp.float32),
        in_specs=[
            pl.BlockSpec(memory_space=pltpu.MemorySpace.HBM),
            pl.BlockSpec(memory_space=pltpu.MemorySpace.HBM),
        ],
        out_specs=pl.BlockSpec(memory_space=pltpu.VMEM),
        scratch_shapes=[
            pltpu.VMEM((m_total, k_per), jnp.float32),
            pltpu.VMEM((m_total, k_per), jnp.bfloat16),
            pltpu.VMEM((k_total, n), jnp.float32),
            pltpu.VMEM((N_DEV - 1, m_per, k_per), jnp.bfloat16),
            pltpu.SemaphoreType.DMA((N_DEV - 1,)),
            pltpu.SemaphoreType.DMA((N_DEV - 1,)),
            pltpu.SemaphoreType.REGULAR((N_DEV - 1,)),
            pltpu.SemaphoreType.DMA,
            pltpu.SemaphoreType.DMA,
        ],
        compiler_params=pltpu.CompilerParams(
            collective_id=0, vmem_limit_bytes=100 * 1024 * 1024
        ),
    )(x, w_mat)


# device time: 49938 ns/iter; 1.0096x vs baseline; 1.0096x over previous
import jax
import jax.numpy as jnp
from jax import lax
from jax.experimental import pallas as pl
from jax.experimental.pallas import tpu as pltpu

N_DEV = 8


def kernel(x, w_mat):
    m_total, k_per = x.shape
    k_total, n = w_mat.shape
    m_per = m_total // N_DEV

    def body(
        x_hbm, w_hbm, out_ref,
        x_vm, x_bf, w_vm, recv_buf,
        send_sems, recv_sems, ready_sems, sem_x, sem_w,
    ):
        my = lax.axis_index("i")
        my4 = lax.rem(my, 4)
        base = my - my4
        obase = 4 - base

        barrier_sem = pltpu.get_barrier_semaphore()
        pl.semaphore_signal(
            barrier_sem, inc=1,
            device_id=(my,), device_id_type=pl.DeviceIdType.MESH,
        )
        pl.semaphore_wait(barrier_sem, 1)

        cp_x = pltpu.make_async_copy(x_hbm, x_vm, sem_x)
        cp_x.start()
        cp_w = pltpu.make_async_copy(w_hbm, w_vm, sem_w)
        cp_w.start()

        for t in range(1, 4):
            q = base + lax.rem(my4 + t, 4)
            pl.semaphore_signal(
                ready_sems.at[3 - t], inc=1,
                device_id=(q,), device_id_type=pl.DeviceIdType.MESH,
            )
        for t in range(4):
            q = obase + lax.rem(my4 + t, 4)
            pl.semaphore_signal(
                ready_sems.at[3 + (4 - t) % 4], inc=1,
                device_id=(q,), device_id_type=pl.DeviceIdType.MESH,
            )

        cp_x.wait()
        x_bf[:, :] = x_vm[:, :].astype(jnp.bfloat16)

        rdmas = {}

        def send_to(q, dst_slot, gate_slot):
            pl.semaphore_wait(ready_sems.at[gate_slot], 1)
            rdma = pltpu.make_async_remote_copy(
                src_ref=x_bf.at[pl.ds(q * m_per, m_per), :],
                dst_ref=recv_buf.at[dst_slot],
                send_sem=send_sems.at[dst_slot],
                recv_sem=recv_sems.at[dst_slot],
                device_id=(q,),
                device_id_type=pl.DeviceIdType.MESH,
            )
            rdma.start()
            rdmas[dst_slot] = rdma

        for t in range(1, 4):
            q = base + lax.rem(my4 + t, 4)
            send_to(q, 3 - t, t - 1)
        for t in range(4):
            q = obase + lax.rem(my4 + t, 4)
            send_to(q, 3 + (4 - t) % 4, 3 + t)

        cp_w.wait()
        out_ref[:, :] = jnp.dot(
            x_bf[pl.ds(my * m_per, m_per), :],
            w_vm[pl.ds(my * k_per, k_per), :].astype(jnp.bfloat16),
            preferred_element_type=jnp.float32,
        )

        for s in range(N_DEV - 1):
            if s < 3:
                src = base + lax.rem(my4 + s + 1, 4)
            else:
                src = obase + lax.rem(my4 + (s - 3), 4)
            rdmas[s].wait_recv()
            part = jnp.dot(
                recv_buf[s],
                w_vm[pl.ds(src * k_per, k_per), :].astype(jnp.bfloat16),
                preferred_element_type=jnp.float32,
            )
            if s < N_DEV - 2:
                out_ref[:, :] += part
            else:
                y = out_ref[:, :] + part
                out_ref[:, :] = y / (1.0 + jnp.exp(-y))

        for s in range(N_DEV - 1):
            rdmas[s].wait_send()

    return pl.pallas_call(
        body,
        out_shape=jax.ShapeDtypeStruct((m_per, n), jnp.float32),
        in_specs=[
            pl.BlockSpec(memory_space=pltpu.MemorySpace.HBM),
            pl.BlockSpec(memory_space=pltpu.MemorySpace.HBM),
        ],
        out_specs=pl.BlockSpec(memory_space=pltpu.VMEM),
        scratch_shapes=[
            pltpu.VMEM((m_total, k_per), jnp.float32),
            pltpu.VMEM((m_total, k_per), jnp.bfloat16),
            pltpu.VMEM((k_total, n), jnp.float32),
            pltpu.VMEM((N_DEV - 1, m_per, k_per), jnp.bfloat16),
            pltpu.SemaphoreType.DMA((N_DEV - 1,)),
            pltpu.SemaphoreType.DMA((N_DEV - 1,)),
            pltpu.SemaphoreType.REGULAR((N_DEV - 1,)),
            pltpu.SemaphoreType.DMA,
            pltpu.SemaphoreType.DMA,
        ],
        compiler_params=pltpu.CompilerParams(
            collective_id=0, vmem_limit_bytes=100 * 1024 * 1024
        ),
    )(x, w_mat)
